# baseline (speedup 1.0000x reference)
import functools

import jax
import jax.numpy as jnp
from jax import lax
from jax.experimental import pallas as pl
from jax.experimental.pallas import tpu as pltpu
from jax.experimental.pallas import tpu_sc as plsc

U = 100000
NI = 100000
V = 100000
D = 512
VIS = 2048
L = 83
T = 300
B = 1024

KS = (2, 3, 4, 5)
GW = 128
TP = 512
TPI = TP // 2
LP = 88
KMAX = 5


_NW = 32
_ROWS_PW = (B * LP) // _NW
_CH = 128
_NFULL = _ROWS_PW // _CH
_BPW = B // _NW


def _sc_gather_body(ti, tj, users, items_i, items_j, bri, brj,
                    text_table, user_alpha, user_vis, user_txt,
                    item_alpha, beta_tbl,
                    emb_i, emb_j, ua, tv, tt, ia_i, ia_j, ib_i, ib_j,
                    idx_all, rows_v, idxb_v, rows512_v, rowsbeta_v,
                    sem, gs0, gs1, ws0, ws1):
    nc = plsc.get_sparse_core_info().num_cores
    wid = lax.axis_index("s") * nc + lax.axis_index("c")
    tbase = pl.multiple_of(wid * _ROWS_PW, 8)
    gsem = (gs0, gs1)
    wsem = (ws0, ws1)

    def gather_side(src_idx, dst):
        pltpu.sync_copy(src_idx.at[pl.ds(tbase, _ROWS_PW)], idx_all)
        gc = [None, None]
        wc = [None, None]
        for c in range(_NFULL):
            b = c & 1
            if wc[b] is not None:
                wc[b].wait()
            gc[b] = pltpu.async_copy(
                text_table.at[idx_all.at[pl.ds(c * _CH, _CH)]],
                rows_v.at[b], gsem[b])
            if c > 0:
                pb = (c - 1) & 1
                gc[pb].wait()
                wc[pb] = pltpu.async_copy(
                    rows_v.at[pb],
                    dst.at[pl.ds(tbase + (c - 1) * _CH, _CH)], wsem[pb])
        lb = (_NFULL - 1) & 1
        gc[lb].wait()
        wc[lb] = pltpu.async_copy(
            rows_v.at[lb],
            dst.at[pl.ds(tbase + (_NFULL - 1) * _CH, _CH)], wsem[lb])
        wc[0].wait()
        wc[1].wait()

    gather_side(ti, emb_i)
    gather_side(tj, emb_j)

    bbase = pl.multiple_of(wid * _BPW, 8)

    def gather_rows(idx_src, table, dst):
        pltpu.sync_copy(idx_src.at[pl.ds(bbase, _BPW)], idxb_v)
        pltpu.async_copy(table.at[idxb_v], rows512_v, sem).wait()
        pltpu.sync_copy(rows512_v, dst.at[pl.ds(bbase, _BPW)])

    gather_rows(users, user_alpha, ua)
    gather_rows(users, user_vis, tv)
    gather_rows(users, user_txt, tt)
    gather_rows(items_i, item_alpha, ia_i)
    gather_rows(items_j, item_alpha, ia_j)

    def gather_beta(idx_src, dst):
        pltpu.sync_copy(idx_src.at[pl.ds(bbase, _BPW)], idxb_v)
        pltpu.async_copy(beta_tbl.at[idxb_v], rowsbeta_v, sem).wait()
        pltpu.sync_copy(rowsbeta_v, dst.at[pl.ds(bbase, _BPW)])

    gather_beta(bri, ib_i)
    gather_beta(brj, ib_j)


def _sc_gather(ti, tj, users, items_i, items_j, bri, brj,
               text_table, user_alpha, user_vis, user_txt,
               item_alpha, beta_tbl):
    f32 = jnp.float32
    out_type = (
        jax.ShapeDtypeStruct((B * LP, TPI), jnp.int32),
        jax.ShapeDtypeStruct((B * LP, TPI), jnp.int32),
        jax.ShapeDtypeStruct((B, D), f32),
        jax.ShapeDtypeStruct((B, D), f32),
        jax.ShapeDtypeStruct((B, D), f32),
        jax.ShapeDtypeStruct((B, D), f32),
        jax.ShapeDtypeStruct((B, D), f32),
        jax.ShapeDtypeStruct((B, 128), f32),
        jax.ShapeDtypeStruct((B, 128), f32),
    )
    kern = functools.partial(
        pl.kernel,
        mesh=plsc.VectorSubcoreMesh(core_axis_name="c", subcore_axis_name="s"),
        out_type=out_type,
        scratch_types=[
            pltpu.VMEM((_ROWS_PW,), jnp.int32),
            pltpu.VMEM((2, _CH, TPI), jnp.int32),
            pltpu.VMEM((_BPW,), jnp.int32),
            pltpu.VMEM((_BPW, D), f32),
            pltpu.VMEM((_BPW, 128), f32),
            pltpu.SemaphoreType.DMA,
            pltpu.SemaphoreType.DMA,
            pltpu.SemaphoreType.DMA,
            pltpu.SemaphoreType.DMA,
            pltpu.SemaphoreType.DMA,
        ],
    )(_sc_gather_body)
    return kern(ti, tj, users, items_i, items_j, bri, brj, text_table,
                user_alpha, user_vis, user_txt, item_alpha, beta_tbl)


_VB = 800


def _pack_body(src, dst):
    x = src[...]
    xp = jnp.concatenate(
        [x, jnp.zeros((_VB, TP - T), jnp.bfloat16)], axis=1)
    dst[...] = pltpu.bitcast(xp.reshape(2 * _VB, TPI), jnp.int32)


def _pack_table(table):
    return pl.pallas_call(
        _pack_body,
        grid=(V // _VB,),
        in_specs=[pl.BlockSpec((_VB, T), lambda i: (i, 0))],
        out_specs=pl.BlockSpec((_VB, TPI), lambda i: (i, 0)),
        out_shape=jax.ShapeDtypeStruct((V, TPI), jnp.int32),
    )(table.astype(jnp.bfloat16))


_BB = 16
_GRID = B // _BB


def _tc_body(emb_i, emb_j, vf_i, vf_j, ua, ia_i, ia_j, tv, tt,
             ibr_i, ibr_j, ci, cj, wall, bcat, tw, tb, vw, vb, out):
    f32 = jnp.float32

    pidx = lax.broadcasted_iota(jnp.int32, (LP, GW), 0)

    def txt_branch(emb_ref):
        nrow = _BB * LP
        emb = pltpu.bitcast(
            emb_ref[...], jnp.bfloat16).reshape(nrow, TP)
        accs = [None] * 4
        for dk in range(KMAX):
            if dk:
                es = jnp.concatenate(
                    [emb[dk:, :], jnp.zeros((dk, TP), jnp.bfloat16)], axis=0)
            else:
                es = emb
            gmin = max(0, dk - 1)
            off = gmin * GW
            wseg = wall[pl.ds(dk * TP, TP), pl.ds(off, 4 * GW - off)]
            z = lax.dot_general(es, wseg, (((1,), (0,)), ((), ())),
                                preferred_element_type=f32)
            for g in range(gmin, 4):
                zg = z[:, (g - gmin) * GW:(g - gmin + 1) * GW]
                accs[g] = zg if accs[g] is None else accs[g] + zg
        ms = []
        for g in range(4):
            y3 = accs[g].reshape(_BB, LP, GW)
            valid = pidx <= (L - 2) - g
            ms.append(jnp.max(jnp.where(valid[None], y3, -1e30), axis=1))
        m = jnp.concatenate(ms, axis=-1)
        h = jax.nn.sigmoid(m + bcat[...])
        return jax.nn.sigmoid(
            lax.dot_general(h, tw[...], (((1,), (0,)), ((), ())),
                            preferred_element_type=f32) + tb[...])

    txt_i = txt_branch(emb_i)
    txt_j = txt_branch(emb_j)

    def vis_branch(vf_ref):
        return jax.nn.sigmoid(
            lax.dot_general(vf_ref[...].astype(jnp.bfloat16), vw[...],
                            (((1,), (0,)), ((), ())),
                            preferred_element_type=f32) + vb[...])

    vis_i = vis_branch(vf_i)
    vis_j = vis_branch(vf_j)

    lane = lax.broadcasted_iota(jnp.int32, (_BB, 128), 1)
    bi = jnp.sum(jnp.where(lane == ci[...], ibr_i[...], 0.0),
                 axis=-1, keepdims=True)
    bj = jnp.sum(jnp.where(lane == cj[...], ibr_j[...], 0.0),
                 axis=-1, keepdims=True)
    s = bi - bj
    s = s + jnp.sum(ua[...] * (ia_i[...] - ia_j[...]), axis=-1, keepdims=True)
    s = s + jnp.sum(tv[...] * (vis_i - vis_j), axis=-1, keepdims=True)
    s = s + jnp.sum(tt[...] * (txt_i - txt_j), axis=-1, keepdims=True)
    out[...] = s


def _tc_compute(emb_i, emb_j, vf_i, vf_j, ua, ia_i, ia_j, tv, tt,
                ibr_i, ibr_j, ci, cj, wall, bcat, tw, tb, vw, vb):
    f32 = jnp.float32
    row_blk = lambda r, c: pl.BlockSpec((r, c), lambda i: (i, 0))
    full_blk = lambda r, c: pl.BlockSpec((r, c), lambda i: (0, 0))
    return pl.pallas_call(
        _tc_body,
        grid=(_GRID,),
        in_specs=[
            row_blk(_BB * LP, TPI),
            row_blk(_BB * LP, TPI),
            row_blk(_BB, VIS),
            row_blk(_BB, VIS),
            row_blk(_BB, D),
            row_blk(_BB, D),
            row_blk(_BB, D),
            row_blk(_BB, D),
            row_blk(_BB, D),
            row_blk(_BB, 128),
            row_blk(_BB, 128),
            row_blk(_BB, 1),
            row_blk(_BB, 1),
            full_blk(KMAX * TP, 4 * GW),
            full_blk(1, 4 * GW),
            full_blk(4 * GW, D),
            full_blk(1, D),
            full_blk(VIS, D),
            full_blk(1, D),
        ],
        out_specs=row_blk(_BB, 1),
        out_shape=jax.ShapeDtypeStruct((B, 1), f32),
    )(emb_i, emb_j, vf_i, vf_j, ua, ia_i, ia_j, tv, tt, ibr_i, ibr_j, ci, cj,
      wall, bcat, tw, tb, vw, vb)


def _prep_weights(conv_Ws, conv_bs, textnn_W, textnn_b, vis_b):
    f32 = jnp.float32
    wall = jnp.zeros((KMAX * TP, 4 * GW), f32)
    for c, (k, wk) in enumerate(zip(KS, conv_Ws)):
        for dk in range(k):
            wall = wall.at[dk * TP:dk * TP + T, c * GW:c * GW + 100].set(
                wk[:, 0, dk, :].T)
    wall = wall.astype(jnp.bfloat16)
    bcat = jnp.zeros((1, 4 * GW), f32)
    tw = jnp.zeros((4 * GW, D), f32)
    for c, bk in enumerate(conv_bs):
        bcat = bcat.at[0, c * GW:c * GW + 100].set(bk)
        tw = tw.at[c * GW:c * GW + 100, :].set(textnn_W[c * 100:(c + 1) * 100, :])
    return wall, bcat, tw, textnn_b.reshape(1, D), vis_b.reshape(1, D)


def _prep_visw(vis_W):
    return vis_W.astype(jnp.bfloat16)


def kernel(users, items_i, items_j, visfeat_i, visfeat_j, text_i, text_j,
           user_alpha, item_alpha, user_beta, item_beta, user_visembed,
           user_textembed, vis_W, vis_b, text_table, conv_W2, conv_b2,
           conv_W3, conv_b3, conv_W4, conv_b4, conv_W5, conv_b5,
           textnn_W, textnn_b):
    del user_beta
    i32 = jnp.int32
    ti2 = text_i.astype(i32)
    tj2 = text_j.astype(i32)
    ti = jnp.concatenate([ti2, ti2[:, :LP - L]], axis=1).reshape(-1)
    tj = jnp.concatenate([tj2, tj2[:, :LP - L]], axis=1).reshape(-1)
    u = users.astype(i32)
    ii = items_i.astype(i32)
    ij = items_j.astype(i32)

    table_p = _pack_table(text_table)
    beta_tbl = jnp.pad(item_beta.reshape(-1), (0, 782 * 128 - NI)).reshape(782, 128)
    bri = jax.lax.shift_right_logical(ii, 7)
    brj = jax.lax.shift_right_logical(ij, 7)
    ci = jax.lax.bitwise_and(ii, 127).reshape(B, 1)
    cj = jax.lax.bitwise_and(ij, 127).reshape(B, 1)
    (emb_i, emb_j, ua, tv, tt, ia_i, ia_j, ibr_i, ibr_j) = _sc_gather(
        ti, tj, u, ii, ij, bri, brj, table_p, user_alpha, user_visembed,
        user_textembed, item_alpha, beta_tbl)

    wall, bcat, tw, tb, vb = _prep_weights(
        (conv_W2, conv_W3, conv_W4, conv_W5),
        (conv_b2, conv_b3, conv_b4, conv_b5),
        textnn_W, textnn_b, vis_b)

    out = _tc_compute(emb_i, emb_j, visfeat_i, visfeat_j, ua, ia_i, ia_j,
                      tv, tt, ibr_i, ibr_j, ci, cj, wall, bcat, tw, tb,
                      _prep_visw(vis_W), vb)
    return out.reshape(B)


# speedup vs baseline: 1.0483x; 1.0483x over previous
import functools

import jax
import jax.numpy as jnp
from jax import lax
from jax.experimental import pallas as pl
from jax.experimental.pallas import tpu as pltpu
from jax.experimental.pallas import tpu_sc as plsc

U = 100000
NI = 100000
V = 100000
D = 512
VIS = 2048
L = 83
T = 300
B = 1024

KS = (2, 3, 4, 5)
GW = 128
TP = 512
TPI = TP // 2
LP = 88
KMAX = 5


_NW = 32
_ROWS_PW = (B * LP) // _NW
_CH = 128
_NFULL = _ROWS_PW // _CH
_BPW = B // _NW


def _sc_gather_body(ti, tj, users, items_i, items_j, bri, brj,
                    text_table, user_alpha, user_vis, user_txt,
                    item_alpha, beta_tbl,
                    emb_i, emb_j, ua, tv, tt, ia_i, ia_j, ib_i, ib_j,
                    idx_all, rows_v, idxb_v, rows512_v, rowsbeta_v,
                    sem, gs0, gs1, ws0, ws1):
    nc = plsc.get_sparse_core_info().num_cores
    wid = lax.axis_index("s") * nc + lax.axis_index("c")
    tbase = pl.multiple_of(wid * _ROWS_PW, 8)
    gsem = (gs0, gs1)
    wsem = (ws0, ws1)

    def gather_side(src_idx, dst):
        pltpu.sync_copy(src_idx.at[pl.ds(tbase, _ROWS_PW)], idx_all)
        gc = [None, None]
        wc = [None, None]
        for c in range(_NFULL):
            b = c & 1
            if wc[b] is not None:
                wc[b].wait()
            gc[b] = pltpu.async_copy(
                text_table.at[idx_all.at[pl.ds(c * _CH, _CH)]],
                rows_v.at[b], gsem[b])
            if c > 0:
                pb = (c - 1) & 1
                gc[pb].wait()
                wc[pb] = pltpu.async_copy(
                    rows_v.at[pb],
                    dst.at[pl.ds(tbase + (c - 1) * _CH, _CH)], wsem[pb])
        lb = (_NFULL - 1) & 1
        gc[lb].wait()
        wc[lb] = pltpu.async_copy(
            rows_v.at[lb],
            dst.at[pl.ds(tbase + (_NFULL - 1) * _CH, _CH)], wsem[lb])
        wc[0].wait()
        wc[1].wait()

    gather_side(ti, emb_i)
    gather_side(tj, emb_j)

    bbase = pl.multiple_of(wid * _BPW, 8)

    def gather_rows(idx_src, table, dst):
        pltpu.sync_copy(idx_src.at[pl.ds(bbase, _BPW)], idxb_v)
        pltpu.async_copy(table.at[idxb_v], rows512_v, sem).wait()
        pltpu.sync_copy(rows512_v, dst.at[pl.ds(bbase, _BPW)])

    gather_rows(users, user_alpha, ua)
    gather_rows(users, user_vis, tv)
    gather_rows(users, user_txt, tt)
    gather_rows(items_i, item_alpha, ia_i)
    gather_rows(items_j, item_alpha, ia_j)

    def gather_beta(idx_src, dst):
        pltpu.sync_copy(idx_src.at[pl.ds(bbase, _BPW)], idxb_v)
        pltpu.async_copy(beta_tbl.at[idxb_v], rowsbeta_v, sem).wait()
        pltpu.sync_copy(rowsbeta_v, dst.at[pl.ds(bbase, _BPW)])

    gather_beta(bri, ib_i)
    gather_beta(brj, ib_j)


def _sc_gather(ti, tj, users, items_i, items_j, bri, brj,
               text_table, user_alpha, user_vis, user_txt,
               item_alpha, beta_tbl):
    f32 = jnp.float32
    out_type = (
        jax.ShapeDtypeStruct((B * LP, TPI), jnp.int32),
        jax.ShapeDtypeStruct((B * LP, TPI), jnp.int32),
        jax.ShapeDtypeStruct((B, D), f32),
        jax.ShapeDtypeStruct((B, D), f32),
        jax.ShapeDtypeStruct((B, D), f32),
        jax.ShapeDtypeStruct((B, D), f32),
        jax.ShapeDtypeStruct((B, D), f32),
        jax.ShapeDtypeStruct((B, 128), f32),
        jax.ShapeDtypeStruct((B, 128), f32),
    )
    kern = functools.partial(
        pl.kernel,
        mesh=plsc.VectorSubcoreMesh(core_axis_name="c", subcore_axis_name="s"),
        out_type=out_type,
        scratch_types=[
            pltpu.VMEM((_ROWS_PW,), jnp.int32),
            pltpu.VMEM((2, _CH, TPI), jnp.int32),
            pltpu.VMEM((_BPW,), jnp.int32),
            pltpu.VMEM((_BPW, D), f32),
            pltpu.VMEM((_BPW, 128), f32),
            pltpu.SemaphoreType.DMA,
            pltpu.SemaphoreType.DMA,
            pltpu.SemaphoreType.DMA,
            pltpu.SemaphoreType.DMA,
            pltpu.SemaphoreType.DMA,
        ],
    )(_sc_gather_body)
    return kern(ti, tj, users, items_i, items_j, bri, brj, text_table,
                user_alpha, user_vis, user_txt, item_alpha, beta_tbl)


_VB = 1000


def _pack_body(src, dst):
    x = src[...].astype(jnp.bfloat16)
    xp = jnp.concatenate(
        [x, jnp.zeros((_VB, TP - T), jnp.bfloat16)], axis=1)
    dst[...] = pltpu.bitcast(xp.reshape(2 * _VB, TPI), jnp.int32)


def _pack_table(table):
    return pl.pallas_call(
        _pack_body,
        grid=(V // _VB,),
        in_specs=[pl.BlockSpec((_VB, T), lambda i: (i, 0))],
        out_specs=pl.BlockSpec((_VB, TPI), lambda i: (i, 0)),
        out_shape=jax.ShapeDtypeStruct((V, TPI), jnp.int32),
    )(table)


_BB = 32
_GRID = B // _BB


def _tc_body(emb_i, emb_j, vf_i, vf_j, ua, ia_i, ia_j, tv, tt,
             ibr_i, ibr_j, ci, cj, wall, bcat, tw, tb, vw, vb, out):
    f32 = jnp.float32

    pidx = lax.broadcasted_iota(jnp.int32, (LP, GW), 0)

    def txt_branch(emb_ref):
        nrow = _BB * LP
        emb = pltpu.bitcast(
            emb_ref[...], jnp.bfloat16).reshape(nrow, TP)
        accs = [None] * 4
        for dk in range(KMAX):
            if dk:
                es = jnp.concatenate(
                    [emb[dk:, :], jnp.zeros((dk, TP), jnp.bfloat16)], axis=0)
            else:
                es = emb
            gmin = max(0, dk - 1)
            off = gmin * GW
            wseg = wall[pl.ds(dk * TP, TP), pl.ds(off, 4 * GW - off)]
            z = lax.dot_general(es, wseg, (((1,), (0,)), ((), ())),
                                preferred_element_type=f32)
            for g in range(gmin, 4):
                zg = z[:, (g - gmin) * GW:(g - gmin + 1) * GW]
                accs[g] = zg if accs[g] is None else accs[g] + zg
        ms = []
        for g in range(4):
            y3 = accs[g].reshape(_BB, LP, GW)
            valid = pidx <= (L - 2) - g
            ms.append(jnp.max(jnp.where(valid[None], y3, -1e30), axis=1))
        m = jnp.concatenate(ms, axis=-1)
        h = jax.nn.sigmoid(m + bcat[...])
        return jax.nn.sigmoid(
            lax.dot_general(h, tw[...], (((1,), (0,)), ((), ())),
                            preferred_element_type=f32) + tb[...])

    txt_i = txt_branch(emb_i)
    txt_j = txt_branch(emb_j)

    def vis_branch(vf_ref):
        return jax.nn.sigmoid(
            lax.dot_general(vf_ref[...].astype(jnp.bfloat16), vw[...],
                            (((1,), (0,)), ((), ())),
                            preferred_element_type=f32) + vb[...])

    vis_i = vis_branch(vf_i)
    vis_j = vis_branch(vf_j)

    lane = lax.broadcasted_iota(jnp.int32, (_BB, 128), 1)
    bi = jnp.sum(jnp.where(lane == ci[...], ibr_i[...], 0.0),
                 axis=-1, keepdims=True)
    bj = jnp.sum(jnp.where(lane == cj[...], ibr_j[...], 0.0),
                 axis=-1, keepdims=True)
    s = bi - bj
    s = s + jnp.sum(ua[...] * (ia_i[...] - ia_j[...]), axis=-1, keepdims=True)
    s = s + jnp.sum(tv[...] * (vis_i - vis_j), axis=-1, keepdims=True)
    s = s + jnp.sum(tt[...] * (txt_i - txt_j), axis=-1, keepdims=True)
    out[...] = s


def _tc_compute(emb_i, emb_j, vf_i, vf_j, ua, ia_i, ia_j, tv, tt,
                ibr_i, ibr_j, ci, cj, wall, bcat, tw, tb, vw, vb):
    f32 = jnp.float32
    row_blk = lambda r, c: pl.BlockSpec((r, c), lambda i: (i, 0))
    full_blk = lambda r, c: pl.BlockSpec((r, c), lambda i: (0, 0))
    return pl.pallas_call(
        _tc_body,
        grid=(_GRID,),
        in_specs=[
            row_blk(_BB * LP, TPI),
            row_blk(_BB * LP, TPI),
            row_blk(_BB, VIS),
            row_blk(_BB, VIS),
            row_blk(_BB, D),
            row_blk(_BB, D),
            row_blk(_BB, D),
            row_blk(_BB, D),
            row_blk(_BB, D),
            row_blk(_BB, 128),
            row_blk(_BB, 128),
            row_blk(_BB, 1),
            row_blk(_BB, 1),
            full_blk(KMAX * TP, 4 * GW),
            full_blk(1, 4 * GW),
            full_blk(4 * GW, D),
            full_blk(1, D),
            full_blk(VIS, D),
            full_blk(1, D),
        ],
        out_specs=row_blk(_BB, 1),
        out_shape=jax.ShapeDtypeStruct((B, 1), f32),
    )(emb_i, emb_j, vf_i, vf_j, ua, ia_i, ia_j, tv, tt, ibr_i, ibr_j, ci, cj,
      wall, bcat, tw, tb, vw, vb)


def _prep_weights(conv_Ws, conv_bs, textnn_W, textnn_b, vis_b):
    f32 = jnp.float32
    wall = jnp.zeros((KMAX * TP, 4 * GW), f32)
    for c, (k, wk) in enumerate(zip(KS, conv_Ws)):
        for dk in range(k):
            wall = wall.at[dk * TP:dk * TP + T, c * GW:c * GW + 100].set(
                wk[:, 0, dk, :].T)
    wall = wall.astype(jnp.bfloat16)
    bcat = jnp.zeros((1, 4 * GW), f32)
    tw = jnp.zeros((4 * GW, D), f32)
    for c, bk in enumerate(conv_bs):
        bcat = bcat.at[0, c * GW:c * GW + 100].set(bk)
        tw = tw.at[c * GW:c * GW + 100, :].set(textnn_W[c * 100:(c + 1) * 100, :])
    return wall, bcat, tw, textnn_b.reshape(1, D), vis_b.reshape(1, D)


def _prep_visw(vis_W):
    return vis_W.astype(jnp.bfloat16)


def kernel(users, items_i, items_j, visfeat_i, visfeat_j, text_i, text_j,
           user_alpha, item_alpha, user_beta, item_beta, user_visembed,
           user_textembed, vis_W, vis_b, text_table, conv_W2, conv_b2,
           conv_W3, conv_b3, conv_W4, conv_b4, conv_W5, conv_b5,
           textnn_W, textnn_b):
    del user_beta
    i32 = jnp.int32
    ti2 = text_i.astype(i32)
    tj2 = text_j.astype(i32)
    ti = jnp.concatenate([ti2, ti2[:, :LP - L]], axis=1).reshape(-1)
    tj = jnp.concatenate([tj2, tj2[:, :LP - L]], axis=1).reshape(-1)
    u = users.astype(i32)
    ii = items_i.astype(i32)
    ij = items_j.astype(i32)

    table_p = _pack_table(text_table)
    beta_tbl = jnp.pad(item_beta.reshape(-1), (0, 782 * 128 - NI)).reshape(782, 128)
    bri = jax.lax.shift_right_logical(ii, 7)
    brj = jax.lax.shift_right_logical(ij, 7)
    ci = jax.lax.bitwise_and(ii, 127).reshape(B, 1)
    cj = jax.lax.bitwise_and(ij, 127).reshape(B, 1)
    (emb_i, emb_j, ua, tv, tt, ia_i, ia_j, ibr_i, ibr_j) = _sc_gather(
        ti, tj, u, ii, ij, bri, brj, table_p, user_alpha, user_visembed,
        user_textembed, item_alpha, beta_tbl)

    wall, bcat, tw, tb, vb = _prep_weights(
        (conv_W2, conv_W3, conv_W4, conv_W5),
        (conv_b2, conv_b3, conv_b4, conv_b5),
        textnn_W, textnn_b, vis_b)

    out = _tc_compute(emb_i, emb_j, visfeat_i, visfeat_j, ua, ia_i, ia_j,
                      tv, tt, ibr_i, ibr_j, ci, cj, wall, bcat, tw, tb,
                      _prep_visw(vis_W), vb)
    return out.reshape(B)


# speedup vs baseline: 1.0523x; 1.0038x over previous
import functools

import jax
import jax.numpy as jnp
from jax import lax
from jax.experimental import pallas as pl
from jax.experimental.pallas import tpu as pltpu
from jax.experimental.pallas import tpu_sc as plsc

U = 100000
NI = 100000
V = 100000
D = 512
VIS = 2048
L = 83
T = 300
B = 1024

KS = (2, 3, 4, 5)
GW = 128
TP = 512
TPI = TP // 2
LP = 88
KMAX = 5


_NW = 32
_ROWS_PW = (B * LP) // _NW
_CH = 128
_NFULL = _ROWS_PW // _CH
_BPW = B // _NW


def _sc_gather_body(ti, tj, users, items_i, items_j, bri, brj,
                    text_table, user_alpha, user_vis, user_txt,
                    item_alpha, beta_tbl,
                    emb_i, emb_j, ua, tv, tt, ia_i, ia_j, ib_i, ib_j,
                    idx_all, rows_v, idxb_v, rows512_v, rowsbeta_v,
                    sem, gs0, gs1, ws0, ws1):
    nc = plsc.get_sparse_core_info().num_cores
    wid = lax.axis_index("s") * nc + lax.axis_index("c")
    tbase = pl.multiple_of(wid * _ROWS_PW, 8)
    gsem = (gs0, gs1)
    wsem = (ws0, ws1)

    def gather_side(src_idx, dst):
        pltpu.sync_copy(src_idx.at[pl.ds(tbase, _ROWS_PW)], idx_all)
        gc = [None, None]
        wc = [None, None]
        for c in range(_NFULL):
            b = c & 1
            if wc[b] is not None:
                wc[b].wait()
            gc[b] = pltpu.async_copy(
                text_table.at[idx_all.at[pl.ds(c * _CH, _CH)]],
                rows_v.at[b], gsem[b])
            if c > 0:
                pb = (c - 1) & 1
                gc[pb].wait()
                wc[pb] = pltpu.async_copy(
                    rows_v.at[pb],
                    dst.at[pl.ds(tbase + (c - 1) * _CH, _CH)], wsem[pb])
        lb = (_NFULL - 1) & 1
        gc[lb].wait()
        wc[lb] = pltpu.async_copy(
            rows_v.at[lb],
            dst.at[pl.ds(tbase + (_NFULL - 1) * _CH, _CH)], wsem[lb])
        wc[0].wait()
        wc[1].wait()

    gather_side(ti, emb_i)
    gather_side(tj, emb_j)

    bbase = pl.multiple_of(wid * _BPW, 8)

    def gather_rows(idx_src, table, dst):
        pltpu.sync_copy(idx_src.at[pl.ds(bbase, _BPW)], idxb_v)
        pltpu.async_copy(table.at[idxb_v], rows512_v, sem).wait()
        pltpu.sync_copy(rows512_v, dst.at[pl.ds(bbase, _BPW)])

    gather_rows(users, user_alpha, ua)
    gather_rows(users, user_vis, tv)
    gather_rows(users, user_txt, tt)
    gather_rows(items_i, item_alpha, ia_i)
    gather_rows(items_j, item_alpha, ia_j)

    def gather_beta(idx_src, dst):
        pltpu.sync_copy(idx_src.at[pl.ds(bbase, _BPW)], idxb_v)
        pltpu.async_copy(beta_tbl.at[idxb_v], rowsbeta_v, sem).wait()
        pltpu.sync_copy(rowsbeta_v, dst.at[pl.ds(bbase, _BPW)])

    gather_beta(bri, ib_i)
    gather_beta(brj, ib_j)


def _sc_gather(ti, tj, users, items_i, items_j, bri, brj,
               text_table, user_alpha, user_vis, user_txt,
               item_alpha, beta_tbl):
    f32 = jnp.float32
    out_type = (
        jax.ShapeDtypeStruct((B * LP, TPI), jnp.int32),
        jax.ShapeDtypeStruct((B * LP, TPI), jnp.int32),
        jax.ShapeDtypeStruct((B, D), f32),
        jax.ShapeDtypeStruct((B, D), f32),
        jax.ShapeDtypeStruct((B, D), f32),
        jax.ShapeDtypeStruct((B, D), f32),
        jax.ShapeDtypeStruct((B, D), f32),
        jax.ShapeDtypeStruct((B, 128), f32),
        jax.ShapeDtypeStruct((B, 128), f32),
    )
    kern = functools.partial(
        pl.kernel,
        mesh=plsc.VectorSubcoreMesh(core_axis_name="c", subcore_axis_name="s"),
        out_type=out_type,
        scratch_types=[
            pltpu.VMEM((_ROWS_PW,), jnp.int32),
            pltpu.VMEM((2, _CH, TPI), jnp.int32),
            pltpu.VMEM((_BPW,), jnp.int32),
            pltpu.VMEM((_BPW, D), f32),
            pltpu.VMEM((_BPW, 128), f32),
            pltpu.SemaphoreType.DMA,
            pltpu.SemaphoreType.DMA,
            pltpu.SemaphoreType.DMA,
            pltpu.SemaphoreType.DMA,
            pltpu.SemaphoreType.DMA,
        ],
    )(_sc_gather_body)
    return kern(ti, tj, users, items_i, items_j, bri, brj, text_table,
                user_alpha, user_vis, user_txt, item_alpha, beta_tbl)


_VB = 1000


def _pack_body(src, dst):
    x = src[...].astype(jnp.bfloat16)
    xp = jnp.concatenate(
        [x, jnp.zeros((_VB, TP - T), jnp.bfloat16)], axis=1)
    dst[...] = pltpu.bitcast(xp.reshape(2 * _VB, TPI), jnp.int32)


def _pack_table(table):
    return pl.pallas_call(
        _pack_body,
        grid=(V // _VB,),
        in_specs=[pl.BlockSpec((_VB, T), lambda i: (i, 0))],
        out_specs=pl.BlockSpec((_VB, TPI), lambda i: (i, 0)),
        out_shape=jax.ShapeDtypeStruct((V, TPI), jnp.int32),
    )(table)


_BB = 64
_GRID = B // _BB


def _tc_body(emb_i, emb_j, vf_i, vf_j, ua, ia_i, ia_j, tv, tt,
             ibr_i, ibr_j, ci, cj, wall, bcat, tw, tb, vw, vb, out):
    f32 = jnp.float32

    pidx = lax.broadcasted_iota(jnp.int32, (LP, GW), 0)

    def txt_branch(emb_ref):
        nrow = _BB * LP
        emb = pltpu.bitcast(
            emb_ref[...], jnp.bfloat16).reshape(nrow, TP)
        accs = [None] * 4
        for dk in range(KMAX):
            if dk:
                es = jnp.concatenate(
                    [emb[dk:, :], jnp.zeros((dk, TP), jnp.bfloat16)], axis=0)
            else:
                es = emb
            gmin = max(0, dk - 1)
            off = gmin * GW
            wseg = wall[pl.ds(dk * TP, TP), pl.ds(off, 4 * GW - off)]
            z = lax.dot_general(es, wseg, (((1,), (0,)), ((), ())),
                                preferred_element_type=f32)
            for g in range(gmin, 4):
                zg = z[:, (g - gmin) * GW:(g - gmin + 1) * GW]
                accs[g] = zg if accs[g] is None else accs[g] + zg
        ms = []
        for g in range(4):
            y3 = accs[g].reshape(_BB, LP, GW)
            valid = pidx <= (L - 2) - g
            ms.append(jnp.max(jnp.where(valid[None], y3, -1e30), axis=1))
        m = jnp.concatenate(ms, axis=-1)
        h = jax.nn.sigmoid(m + bcat[...])
        return jax.nn.sigmoid(
            lax.dot_general(h, tw[...], (((1,), (0,)), ((), ())),
                            preferred_element_type=f32) + tb[...])

    txt_i = txt_branch(emb_i)
    txt_j = txt_branch(emb_j)

    def vis_branch(vf_ref):
        return jax.nn.sigmoid(
            lax.dot_general(vf_ref[...].astype(jnp.bfloat16), vw[...],
                            (((1,), (0,)), ((), ())),
                            preferred_element_type=f32) + vb[...])

    vis_i = vis_branch(vf_i)
    vis_j = vis_branch(vf_j)

    lane = lax.broadcasted_iota(jnp.int32, (_BB, 128), 1)
    bi = jnp.sum(jnp.where(lane == ci[...], ibr_i[...], 0.0),
                 axis=-1, keepdims=True)
    bj = jnp.sum(jnp.where(lane == cj[...], ibr_j[...], 0.0),
                 axis=-1, keepdims=True)
    s = bi - bj
    s = s + jnp.sum(ua[...] * (ia_i[...] - ia_j[...]), axis=-1, keepdims=True)
    s = s + jnp.sum(tv[...] * (vis_i - vis_j), axis=-1, keepdims=True)
    s = s + jnp.sum(tt[...] * (txt_i - txt_j), axis=-1, keepdims=True)
    out[...] = s


def _tc_compute(emb_i, emb_j, vf_i, vf_j, ua, ia_i, ia_j, tv, tt,
                ibr_i, ibr_j, ci, cj, wall, bcat, tw, tb, vw, vb):
    f32 = jnp.float32
    row_blk = lambda r, c: pl.BlockSpec((r, c), lambda i: (i, 0))
    full_blk = lambda r, c: pl.BlockSpec((r, c), lambda i: (0, 0))
    return pl.pallas_call(
        _tc_body,
        grid=(_GRID,),
        in_specs=[
            row_blk(_BB * LP, TPI),
            row_blk(_BB * LP, TPI),
            row_blk(_BB, VIS),
            row_blk(_BB, VIS),
            row_blk(_BB, D),
            row_blk(_BB, D),
            row_blk(_BB, D),
            row_blk(_BB, D),
            row_blk(_BB, D),
            row_blk(_BB, 128),
            row_blk(_BB, 128),
            row_blk(_BB, 1),
            row_blk(_BB, 1),
            full_blk(KMAX * TP, 4 * GW),
            full_blk(1, 4 * GW),
            full_blk(4 * GW, D),
            full_blk(1, D),
            full_blk(VIS, D),
            full_blk(1, D),
        ],
        out_specs=row_blk(_BB, 1),
        out_shape=jax.ShapeDtypeStruct((B, 1), f32),
    )(emb_i, emb_j, vf_i, vf_j, ua, ia_i, ia_j, tv, tt, ibr_i, ibr_j, ci, cj,
      wall, bcat, tw, tb, vw, vb)


def _prep_weights(conv_Ws, conv_bs, textnn_W, textnn_b, vis_b):
    f32 = jnp.float32
    wall = jnp.zeros((KMAX * TP, 4 * GW), f32)
    for c, (k, wk) in enumerate(zip(KS, conv_Ws)):
        for dk in range(k):
            wall = wall.at[dk * TP:dk * TP + T, c * GW:c * GW + 100].set(
                wk[:, 0, dk, :].T)
    wall = wall.astype(jnp.bfloat16)
    bcat = jnp.zeros((1, 4 * GW), f32)
    tw = jnp.zeros((4 * GW, D), f32)
    for c, bk in enumerate(conv_bs):
        bcat = bcat.at[0, c * GW:c * GW + 100].set(bk)
        tw = tw.at[c * GW:c * GW + 100, :].set(textnn_W[c * 100:(c + 1) * 100, :])
    return wall, bcat, tw, textnn_b.reshape(1, D), vis_b.reshape(1, D)


def _prep_visw(vis_W):
    return vis_W.astype(jnp.bfloat16)


def kernel(users, items_i, items_j, visfeat_i, visfeat_j, text_i, text_j,
           user_alpha, item_alpha, user_beta, item_beta, user_visembed,
           user_textembed, vis_W, vis_b, text_table, conv_W2, conv_b2,
           conv_W3, conv_b3, conv_W4, conv_b4, conv_W5, conv_b5,
           textnn_W, textnn_b):
    del user_beta
    i32 = jnp.int32
    ti2 = text_i.astype(i32)
    tj2 = text_j.astype(i32)
    ti = jnp.concatenate([ti2, ti2[:, :LP - L]], axis=1).reshape(-1)
    tj = jnp.concatenate([tj2, tj2[:, :LP - L]], axis=1).reshape(-1)
    u = users.astype(i32)
    ii = items_i.astype(i32)
    ij = items_j.astype(i32)

    table_p = _pack_table(text_table)
    beta_tbl = jnp.pad(item_beta.reshape(-1), (0, 782 * 128 - NI)).reshape(782, 128)
    bri = jax.lax.shift_right_logical(ii, 7)
    brj = jax.lax.shift_right_logical(ij, 7)
    ci = jax.lax.bitwise_and(ii, 127).reshape(B, 1)
    cj = jax.lax.bitwise_and(ij, 127).reshape(B, 1)
    (emb_i, emb_j, ua, tv, tt, ia_i, ia_j, ibr_i, ibr_j) = _sc_gather(
        ti, tj, u, ii, ij, bri, brj, table_p, user_alpha, user_visembed,
        user_textembed, item_alpha, beta_tbl)

    wall, bcat, tw, tb, vb = _prep_weights(
        (conv_W2, conv_W3, conv_W4, conv_W5),
        (conv_b2, conv_b3, conv_b4, conv_b5),
        textnn_W, textnn_b, vis_b)

    out = _tc_compute(emb_i, emb_j, visfeat_i, visfeat_j, ua, ia_i, ia_j,
                      tv, tt, ibr_i, ibr_j, ci, cj, wall, bcat, tw, tb,
                      _prep_visw(vis_W), vb)
    return out.reshape(B)
